# Initial kernel scaffold; baseline (speedup 1.0000x reference)
import jax
import jax.numpy as jnp
from jax.experimental import pallas as pl


def kernel(pred_logits, pred_boxes, pred_vectors, pred_interms, pred_actions, target_sizes):
    raise NotImplementedError("write your pallas kernel here")


# speedup vs baseline: 1.6548x; 1.6548x over previous
import functools

import jax
import jax.numpy as jnp
from jax.experimental import pallas as pl
from jax.experimental.pallas import tpu as pltpu

_B, _Q, _C = 16, 300, 91
_N = _Q * _C
_NP = 27392
_K = 50
_KP = 64
_MINI32 = -2147483648


def _sortable(u):
    m = jax.lax.shift_right_logical(jax.lax.shift_right_arithmetic(u, 31), 1)
    return jax.lax.bitwise_xor(u, m)


def _body(ik_ref, boxes_ref, interms_ref, pa_ref, ts_ref,
          scores_ref, labels_ref, boxeso_ref, si_ref, li_ref, la_ref,
          xk_ref):
    xk_ref[...] = _sortable(ik_ref[...])

    lane_big = jax.lax.broadcasted_iota(jnp.int32, (_B, _NP), 1)
    lane_k = jax.lax.broadcasted_iota(jnp.int32, (_B, _KP), 1)

    def step(k, carry):
        acc_s, acc_i = carry
        x = xk_ref[...]
        gm = jnp.max(x, axis=1, keepdims=True)
        eq = x == gm
        pos = jnp.min(jnp.where(eq, lane_big, jnp.int32(_NP)), axis=1,
                      keepdims=True)
        xk_ref[...] = jnp.where(lane_big == pos, jnp.int32(_MINI32), x)
        ins = lane_k == k
        acc_s = jnp.where(ins, gm, acc_s)
        acc_i = jnp.where(ins, pos, acc_i)
        return acc_s, acc_i

    init = (jnp.full((_B, _KP), jnp.int32(_MINI32), jnp.int32),
            jnp.zeros((_B, _KP), jnp.int32))
    skeys, idx = jax.lax.fori_loop(0, _K, step, init)

    u = _sortable(skeys)
    logit = jax.lax.bitcast_convert_type(u, jnp.float32)
    scores_ref[...] = jax.nn.sigmoid(logit)

    rows = jax.lax.shift_right_logical(idx * 11523, 20)
    labels_ref[...] = idx - rows * _C

    pi = interms_ref[...]
    rmax = jnp.max(pi, axis=2)
    i117 = jax.lax.broadcasted_iota(jnp.int32, pi.shape, 2)
    rarg = jnp.min(jnp.where(pi == rmax[:, :, None], i117, jnp.int32(1000)),
                   axis=2)

    bx = boxes_ref[...]
    cx, cy, w, h = (bx[..., 0:1], bx[..., 1:2], bx[..., 2:3], bx[..., 3:4])
    zeros = jnp.zeros_like(cx)
    v = jnp.concatenate(
        [cx - 0.5 * w, cy - 0.5 * h, cx + 0.5 * w, cy + 0.5 * h,
         rmax[:, :, None], rarg.astype(jnp.float32)[:, :, None],
         zeros, zeros], axis=-1)

    i300 = jax.lax.broadcasted_iota(jnp.int32, (_KP, _Q), 1)
    gs = []
    for b in range(_B):
        oh = (rows[b][:, None] == i300).astype(jnp.float32)
        gs.append(jax.lax.dot_general(
            oh, v[b], (((1,), (0,)), ((), ())),
            preferred_element_type=jnp.float32))
    g = jnp.stack(gs, axis=0)

    ts = ts_ref[...]
    scale = jnp.concatenate(
        [ts[:, 1:2], ts[:, 0:1], ts[:, 1:2], ts[:, 0:1]], axis=1)
    boxeso_ref[...] = g[:, :, 0:4] * scale[:, None, :]
    si_ref[...] = jax.nn.sigmoid(g[:, :, 4])
    li_ref[...] = g[:, :, 5].astype(jnp.int32)

    pa = pa_ref[...]
    am = jnp.max(pa, axis=1, keepdims=True)
    i10 = jax.lax.broadcasted_iota(jnp.int32, pa.shape, 1)
    la = jnp.min(jnp.where(pa == am, i10, jnp.int32(100)), axis=1,
                 keepdims=True)
    la_ref[...] = la


@jax.jit
def kernel(pred_logits, pred_boxes, pred_vectors, pred_interms, pred_actions,
           target_sizes):
    del pred_vectors
    ik = jax.lax.bitcast_convert_type(pred_logits, jnp.int32).reshape(_B, _N)
    ik = jnp.pad(ik, ((0, 0), (0, _NP - _N)), constant_values=-1)
    pa = pred_actions.reshape(_B, 10)
    ts = target_sizes.astype(jnp.float32)

    out_shape = [
        jax.ShapeDtypeStruct((_B, _KP), jnp.float32),
        jax.ShapeDtypeStruct((_B, _KP), jnp.int32),
        jax.ShapeDtypeStruct((_B, _KP, 4), jnp.float32),
        jax.ShapeDtypeStruct((_B, _KP), jnp.float32),
        jax.ShapeDtypeStruct((_B, _KP), jnp.int32),
        jax.ShapeDtypeStruct((_B, 1), jnp.int32),
    ]
    scores, labels, boxes, si, li, la = pl.pallas_call(
        _body,
        out_shape=out_shape,
        scratch_shapes=[pltpu.VMEM((_B, _NP), jnp.int32)],
    )(ik, pred_boxes, pred_interms, pa, ts)

    return (scores[:, :_K], labels[:, :_K], boxes[:, :_K, :],
            si[:, :_K], li[:, :_K], la[:, 0])
